# baseline (speedup 1.0000x reference)
import jax
import jax.numpy as jnp
import numpy as np
from jax import lax
from jax.experimental import pallas as pl
from jax.experimental.pallas import tpu as pltpu
from jax.experimental.pallas import tpu_sc as plsc

L = 16
F = 2
T = 1 << 19
D = 3
N_PTS = 1048576
OUT_D = 44
MASK = T - 1
P1 = np.int32(np.uint32(2654435761).astype(np.int32))
P2 = np.int32(805459861)

NW = 32
PTS_W = N_PTS // NW
CH = 512
NCH = PTS_W // CH
ILEN = 128
NROW = 8 * CH // ILEN
NG = CH // 16

SINC = (6.28316827, -41.33792977, 81.47313282, -75.09327405, 33.95650071)
COSC = (0.99997108, -19.73279612, 64.7143697, -82.70120228, 46.31009229)
UNR = 2


def _sc_body(x_hbm, xn_hbm, table_hbm, out_hbm, xtmp, xnT, fracb, g0pad,
             idxb, g1pad, gatb, g2pad, outb, gsem):
    del g0pad, g1pad, g2pad
    wid = lax.axis_index("s") * 2 + lax.axis_index("c")
    iota16 = lax.iota(jnp.int32, 16)
    zeros16 = jnp.zeros((16,), jnp.int32)
    ones16 = jnp.full((16,), 1, jnp.int32)

    def full16(v):
        return jnp.full((16,), v, jnp.int32)

    def pass_a_grp(g, resf, lT, dense, Rr):
        off = g * 16
        xs = [xnT[d, pl.ds(off, 16)] for d in range(3)]
        pos = [xv * resf for xv in xs]
        pi = [p.astype(jnp.int32) for p in pos]
        for d in range(3):
            fracb[d, pl.ds(off, 16)] = pos[d] - pi[d].astype(jnp.float32)
        row = g >> 3
        col = (g & 7) * 16
        if dense:
            t0 = pi[0] * (Rr * Rr)
            t1 = t0 + (Rr * Rr)
            u0 = pi[1] * Rr
            u1 = u0 + Rr
            z0 = pi[2] + lT
            z1 = z0 + 1
            for c in range(8):
                idx = ((t1 if c & 1 else t0) + (u1 if c & 2 else u0)
                       + (z1 if c & 4 else z0))
                idxb[row + c * (CH // ILEN), pl.ds(col, 16)] = idx
        else:
            a0 = pi[0]
            a1 = a0 + 1
            m0 = pi[1] * P1
            m1 = m0 + P1
            n0 = pi[2] * P2
            n1 = n0 + P2
            xy = [a0 ^ m0, a1 ^ m0, a0 ^ m1, a1 ^ m1]
            for c in range(8):
                h = xy[c & 3] ^ (n1 if c & 4 else n0)
                idxb[row + c * (CH // ILEN), pl.ds(col, 16)] = (
                    (h & MASK) + lT)

    def pass_b_grp(g, col0):
        off = g * 16
        rows16 = off + iota16
        j0 = g >> 3
        iv = (g & 7) * 16 + iota16
        fr = [fracb[d, pl.ds(off, 16)] for d in range(3)]
        om = [1.0 - f for f in fr]
        wxy = [om[0] * om[1], fr[0] * om[1], om[0] * fr[1], fr[0] * fr[1]]
        acc0 = None
        acc1 = None
        for c in range(8):
            w = wxy[c & 3] * (fr[2] if c & 4 else om[2])
            jv = full16(j0 + c * (CH // ILEN))
            gg0 = plsc.load_gather(gatb, [jv, iv, zeros16])
            gg1 = plsc.load_gather(gatb, [jv, iv, ones16])
            acc0 = w * gg0 if acc0 is None else acc0 + w * gg0
            acc1 = w * gg1 if acc1 is None else acc1 + w * gg1
        plsc.store_scatter(outb, [rows16, full16(col0)], acc0)
        plsc.store_scatter(outb, [rows16, full16(col0 + 1)], acc1)

    def run_level(l, resf, lT, dense, Rr, col0):
        del l

        def abody(g2, _):
            for u in range(UNR):
                pass_a_grp(g2 * UNR + u, resf, lT, dense, Rr)
            return 0

        lax.fori_loop(0, NG // UNR, abody, 0)

        def fire(j, _):
            pltpu.make_async_copy(
                table_hbm.at[idxb.at[j]], gatb.at[j], gsem).start()
            return 0

        lax.fori_loop(0, NROW, fire, 0)

        def drainb(j, _):
            pltpu.make_async_copy(
                table_hbm.at[idxb.at[j]], gatb.at[j], gsem).wait()
            return 0

        lax.fori_loop(0, NROW, drainb, 0)

        def bbody(g2, _):
            for u in range(UNR):
                pass_b_grp(g2 * UNR + u, col0)
            return 0

        lax.fori_loop(0, NG // UNR, bbody, 0)

    def freq_pass():
        def body(g2, _):
            for u in range(UNR):
                g = g2 * UNR + u
                off = g * 16
                rows16 = off + iota16
                for d in range(3):
                    xd = plsc.load_gather(xtmp, [rows16, full16(d)])
                    vh = xd * 0.5
                    w = vh + 0.5
                    wf = w.astype(jnp.int32).astype(jnp.float32)
                    fl = wf - jnp.where(w < wf, 1.0, 0.0).astype(jnp.float32)
                    z = vh - fl
                    q = z * z
                    s1 = z * (SINC[0] + q * (SINC[1] + q * (SINC[2] + q * (
                        SINC[3] + q * SINC[4]))))
                    c1 = COSC[0] + q * (COSC[1] + q * (COSC[2] + q * (
                        COSC[3] + q * COSC[4])))
                    s2 = 2.0 * s1 * c1
                    c2 = 1.0 - 2.0 * s1 * s1
                    plsc.store_scatter(outb, [rows16, full16(d)], s1)
                    plsc.store_scatter(outb, [rows16, full16(3 + d)], c1)
                    plsc.store_scatter(outb, [rows16, full16(6 + d)], s2)
                    plsc.store_scatter(outb, [rows16, full16(9 + d)], c2)
            return 0

        lax.fori_loop(0, NG // UNR, body, 0)

    def chunk(k, _):
        base = wid * PTS_W + k * CH

        pltpu.sync_copy(xn_hbm.at[pl.ds(base, CH), :], xtmp)

        def deint(g2, _):
            for u in range(UNR):
                off = (g2 * UNR + u) * 16
                rows16 = off + iota16
                for d in range(3):
                    v = plsc.load_gather(xtmp, [rows16, full16(d)])
                    xnT[d, pl.ds(off, 16)] = v
            return 0

        lax.fori_loop(0, NG // UNR, deint, 0)
        pltpu.sync_copy(x_hbm.at[pl.ds(base, CH), :], xtmp)
        freq_pass()

        for l in range(3):
            res = 16 << l
            run_level(l, float(res), l * T, True, res + 1, 12 + 2 * l)

        def hash_level(l, _):
            res = jnp.int32(16) << l
            resf = res.astype(jnp.float32)
            lT = l << jnp.int32(19)
            run_level(l, resf, lT, False, 0, 12 + 2 * l)
            return 0

        lax.fori_loop(3, L, hash_level, 0)

        pltpu.sync_copy(outb, out_hbm.at[pl.ds(base, CH), :])
        return 0

    lax.fori_loop(0, NCH, chunk, 0)


def kernel(x, table, bound):
    b = jnp.asarray(bound, jnp.float32)
    xn = jnp.clip((x + b) / (2.0 * b), 0.0, 1.0)
    mesh = plsc.VectorSubcoreMesh(
        core_axis_name="c", subcore_axis_name="s", num_cores=2,
        num_subcores=16)
    fn = pl.kernel(
        _sc_body,
        out_type=jax.ShapeDtypeStruct((N_PTS, OUT_D), jnp.float32),
        mesh=mesh,
        compiler_params=pltpu.CompilerParams(
            needs_layout_passes=False, use_tc_tiling_on_sc=False),
        scratch_types=[
            pltpu.VMEM((CH, D), jnp.float32),
            pltpu.VMEM((D, CH), jnp.float32),
            pltpu.VMEM((D, CH), jnp.float32),
            pltpu.VMEM((512,), jnp.int32),
            pltpu.VMEM((NROW, ILEN), jnp.int32),
            pltpu.VMEM((512,), jnp.int32),
            pltpu.VMEM((NROW, ILEN, F), jnp.float32),
            pltpu.VMEM((512,), jnp.int32),
            pltpu.VMEM((CH, OUT_D), jnp.float32),
            pltpu.SemaphoreType.DMA,
        ],
    )
    return fn(x, xn, table)


# speedup vs baseline: 1.6646x; 1.6646x over previous
import jax
import jax.numpy as jnp
import numpy as np
from jax import lax
from jax.experimental import pallas as pl
from jax.experimental.pallas import tpu as pltpu
from jax.experimental.pallas import tpu_sc as plsc

L = 16
T = 1 << 19
D = 3
N_PTS = 1048576
OUT_D = 44
MASK = T - 1
P1 = np.int32(np.uint32(2654435761).astype(np.int32))
P2 = np.int32(805459861)

NW = 32
PTS_W = N_PTS // NW
CH = 512
NCH = PTS_W // CH
ILEN = 128
NROW = 8 * CH // ILEN
NG = CH // 16
RPC = CH // ILEN

SINC = (6.28316827, -41.33792977, 81.47313282, -75.09327405, 33.95650071)
COSC = (0.99997108, -19.73279612, 64.7143697, -82.70120228, 46.31009229)
UNR = 2


def _sc_body(x_hbm, xnT_hbm, tf0_hbm, tf1_hbm, out_hbm, xtmp, xnT, fracb,
             g0pad, idxb, g1pad, gatb0, gatb1, g2pad, outb, gsem):
    del g0pad, g1pad, g2pad
    wid = lax.axis_index("s") * 2 + lax.axis_index("c")
    iota16 = lax.iota(jnp.int32, 16)

    def full16(v):
        return jnp.full((16,), v, jnp.int32)

    def pass_a_grp(g, resf, lT, dense, Rr):
        off = g * 16
        xs = [xnT[d, pl.ds(off, 16)] for d in range(3)]
        pos = [xv * resf for xv in xs]
        pi = [p.astype(jnp.int32) for p in pos]
        for d in range(3):
            fracb[d, pl.ds(off, 16)] = pos[d] - pi[d].astype(jnp.float32)
        row = g >> 3
        col = (g & 7) * 16
        if dense:
            t0 = pi[0] * (Rr * Rr)
            t1 = t0 + (Rr * Rr)
            u0 = pi[1] * Rr
            u1 = u0 + Rr
            z0 = pi[2] + lT
            z1 = z0 + 1
            for c in range(8):
                idx = ((t1 if c & 1 else t0) + (u1 if c & 2 else u0)
                       + (z1 if c & 4 else z0))
                idxb[row + c * RPC, pl.ds(col, 16)] = idx
        else:
            a0 = pi[0]
            a1 = a0 + 1
            m0 = pi[1] * P1
            m1 = m0 + P1
            n0 = pi[2] * P2
            n1 = n0 + P2
            xy = [a0 ^ m0, a1 ^ m0, a0 ^ m1, a1 ^ m1]
            for c in range(8):
                h = xy[c & 3] ^ (n1 if c & 4 else n0)
                idxb[row + c * RPC, pl.ds(col, 16)] = (h & MASK) + lT

    def pass_b_grp(g, col0):
        off = g * 16
        rows16 = off + iota16
        j0 = g >> 3
        col = (g & 7) * 16
        fr = [fracb[d, pl.ds(off, 16)] for d in range(3)]
        om = [1.0 - f for f in fr]
        wxy = [om[0] * om[1], fr[0] * om[1], om[0] * fr[1], fr[0] * fr[1]]
        acc0 = None
        acc1 = None
        for c in range(8):
            w = wxy[c & 3] * (fr[2] if c & 4 else om[2])
            j = j0 + c * RPC
            gg0 = gatb0[j, pl.ds(col, 16)]
            gg1 = gatb1[j, pl.ds(col, 16)]
            acc0 = w * gg0 if acc0 is None else acc0 + w * gg0
            acc1 = w * gg1 if acc1 is None else acc1 + w * gg1
        plsc.store_scatter(outb, [rows16, full16(col0)], acc0)
        plsc.store_scatter(outb, [rows16, full16(col0 + 1)], acc1)

    def run_level(resf, lT, dense, Rr, col0):
        def abody(g2, _):
            for u in range(UNR):
                pass_a_grp(g2 * UNR + u, resf, lT, dense, Rr)
            return 0

        lax.fori_loop(0, NG // UNR, abody, 0)

        def fire(j, _):
            pltpu.make_async_copy(
                tf0_hbm.at[idxb.at[j]], gatb0.at[j], gsem).start()
            pltpu.make_async_copy(
                tf1_hbm.at[idxb.at[j]], gatb1.at[j], gsem).start()
            return 0

        lax.fori_loop(0, NROW, fire, 0)

        def drainb(j, _):
            pltpu.make_async_copy(
                tf0_hbm.at[idxb.at[j]], gatb0.at[j], gsem).wait()
            pltpu.make_async_copy(
                tf1_hbm.at[idxb.at[j]], gatb1.at[j], gsem).wait()
            return 0

        lax.fori_loop(0, NROW, drainb, 0)

        def bbody(g2, _):
            for u in range(UNR):
                pass_b_grp(g2 * UNR + u, col0)
            return 0

        lax.fori_loop(0, NG // UNR, bbody, 0)

    def freq_pass():
        def body(g2, _):
            for u in range(UNR):
                g = g2 * UNR + u
                off = g * 16
                rows16 = off + iota16
                for d in range(3):
                    xd = plsc.load_gather(xtmp, [rows16, full16(d)])
                    vh = xd * 0.5
                    w = vh + 0.5
                    wf = w.astype(jnp.int32).astype(jnp.float32)
                    fl = wf - jnp.where(w < wf, 1.0, 0.0).astype(jnp.float32)
                    z = vh - fl
                    q = z * z
                    s1 = z * (SINC[0] + q * (SINC[1] + q * (SINC[2] + q * (
                        SINC[3] + q * SINC[4]))))
                    c1 = COSC[0] + q * (COSC[1] + q * (COSC[2] + q * (
                        COSC[3] + q * COSC[4])))
                    s2 = 2.0 * s1 * c1
                    c2 = 1.0 - 2.0 * s1 * s1
                    plsc.store_scatter(outb, [rows16, full16(d)], s1)
                    plsc.store_scatter(outb, [rows16, full16(3 + d)], c1)
                    plsc.store_scatter(outb, [rows16, full16(6 + d)], s2)
                    plsc.store_scatter(outb, [rows16, full16(9 + d)], c2)
            return 0

        lax.fori_loop(0, NG // UNR, body, 0)

    def chunk(k, _):
        base = wid * PTS_W + k * CH

        pltpu.sync_copy(xnT_hbm.at[:, pl.ds(base, CH)], xnT)
        pltpu.sync_copy(x_hbm.at[pl.ds(base, CH), :], xtmp)
        freq_pass()

        for l in range(3):
            res = 16 << l
            run_level(float(res), l * T, True, res + 1, 12 + 2 * l)

        def hash_level(l, _):
            res = jnp.int32(16) << l
            resf = res.astype(jnp.float32)
            lT = l << jnp.int32(19)
            run_level(resf, lT, False, 0, 12 + 2 * l)
            return 0

        lax.fori_loop(3, L, hash_level, 0)

        pltpu.sync_copy(outb, out_hbm.at[pl.ds(base, CH), :])
        return 0

    lax.fori_loop(0, NCH, chunk, 0)


def kernel(x, table, bound):
    b = jnp.asarray(bound, jnp.float32)
    xnT = jnp.clip((x + b) / (2.0 * b), 0.0, 1.0).T
    tf0 = table[:, 0]
    tf1 = table[:, 1]
    mesh = plsc.VectorSubcoreMesh(
        core_axis_name="c", subcore_axis_name="s", num_cores=2,
        num_subcores=16)
    fn = pl.kernel(
        _sc_body,
        out_type=jax.ShapeDtypeStruct((N_PTS, OUT_D), jnp.float32),
        mesh=mesh,
        compiler_params=pltpu.CompilerParams(
            needs_layout_passes=False, use_tc_tiling_on_sc=False),
        scratch_types=[
            pltpu.VMEM((CH, D), jnp.float32),
            pltpu.VMEM((D, CH), jnp.float32),
            pltpu.VMEM((D, CH), jnp.float32),
            pltpu.VMEM((512,), jnp.int32),
            pltpu.VMEM((NROW, ILEN), jnp.int32),
            pltpu.VMEM((512,), jnp.int32),
            pltpu.VMEM((NROW, ILEN), jnp.float32),
            pltpu.VMEM((NROW, ILEN), jnp.float32),
            pltpu.VMEM((512,), jnp.int32),
            pltpu.VMEM((CH, OUT_D), jnp.float32),
            pltpu.SemaphoreType.DMA,
        ],
    )
    return fn(x, xnT, tf0, tf1)
